# Initial kernel scaffold; baseline (speedup 1.0000x reference)
import jax
import jax.numpy as jnp
from jax.experimental import pallas as pl


def kernel(confidence, predicted_locations, labels, gt_locations):
    raise NotImplementedError("write your pallas kernel here")


# speedup vs baseline: 118.9588x; 118.9588x over previous
import functools

import jax
import jax.numpy as jnp
from jax.experimental import pallas as pl
from jax.experimental.pallas import tpu as pltpu

_BETA = 1.0 / 9.0
_NEG_POS_RATIO = 3


def _loss_kernel(c0_ref, c1_ref, lab_ref, p1_ref, p3_ref, g1_ref, g3_ref,
                 out_ref, lmask_ref):
    x = c1_ref[:] - c0_ref[:]
    sp = jnp.maximum(x, 0.0) + jnp.log1p(jnp.exp(-jnp.abs(x)))
    pos = lab_ref[:] > 0
    posf = pos.astype(jnp.float32)
    num_pos = jnp.sum(pos.astype(jnp.int32))
    n_total = lab_ref.shape[0] * lab_ref.shape[1]
    num_neg_avail = n_total - num_pos
    k_eff = jnp.minimum(num_pos * _NEG_POS_RATIO, num_neg_avail)

    lmask_ref[:] = jnp.where(pos, -1.0, sp)

    s_ce_pos = jnp.sum(jnp.where(pos, sp - x, 0.0))

    d1 = jnp.abs(p1_ref[:] - g1_ref[:])
    d3 = jnp.abs(p3_ref[:] - g3_ref[:])
    sl1 = jnp.where(d1 < _BETA, 0.5 / _BETA * d1 * d1, d1 - 0.5 * _BETA) + \
          jnp.where(d3 < _BETA, 0.5 / _BETA * d3 * d3, d3 - 0.5 * _BETA)
    s_sl1 = jnp.sum(sl1 * posf)

    def search_body(i, base):
        cand = base + jax.lax.shift_left(jnp.int32(1), jnp.int32(30) - i)
        keys = jax.lax.bitcast_convert_type(lmask_ref[:], jnp.int32)
        cnt = jnp.sum((keys >= cand).astype(jnp.int32))
        return jnp.where(cnt >= k_eff, cand, base)

    base = jax.lax.fori_loop(0, 31, search_body, jnp.int32(0))

    keys = jax.lax.bitcast_convert_type(lmask_ref[:], jnp.int32)
    gt = keys > base
    count_gt = jnp.sum(gt.astype(jnp.int32))
    s_gt = jnp.sum(jnp.where(gt, lmask_ref[:], 0.0))
    l_thr = jax.lax.bitcast_convert_type(base, jnp.float32)
    remaining = (k_eff - count_gt).astype(jnp.float32)
    s_neg = jnp.where(k_eff == 0, 0.0, s_gt + remaining * l_thr)

    n_sel = (num_pos + k_eff).astype(jnp.float32)
    loss_cls = jnp.clip((s_ce_pos + s_neg) / jnp.maximum(n_sel, 1.0), 0.0, 5.0)
    loss_ver = jnp.clip(
        s_sl1 / jnp.maximum(2.0 * num_pos.astype(jnp.float32), 1.0), 0.0, 5.0)
    loss_total = loss_ver + loss_cls

    row = jax.lax.broadcasted_iota(jnp.int32, (8, 128), 0)
    col = jax.lax.broadcasted_iota(jnp.int32, (8, 128), 1)
    sel0 = (row == 0) & (col == 0)
    sel1 = (row == 0) & (col == 1)
    sel2 = (row == 0) & (col == 2)
    out_ref[:] = (jnp.where(sel0, loss_total, 0.0)
                  + jnp.where(sel1, loss_cls, 0.0)
                  + jnp.where(sel2, loss_ver, 0.0))


@functools.partial(jax.jit, static_argnames=())
def kernel(confidence, predicted_locations, labels, gt_locations):
    c0 = confidence[..., 0]
    c1 = confidence[..., 1]
    p1 = predicted_locations[..., 1]
    p3 = predicted_locations[..., 3]
    g1 = gt_locations[..., 1]
    g3 = gt_locations[..., 3]

    out = pl.pallas_call(
        _loss_kernel,
        out_shape=jax.ShapeDtypeStruct((8, 128), jnp.float32),
        scratch_shapes=[pltpu.VMEM(labels.shape, jnp.float32)],
    )(c0, c1, labels, p1, p3, g1, g3)

    loss_total = out[0, 0]
    loss_cls = out[0, 1]
    loss_ver = out[0, 2]
    loss_refine = jnp.zeros((), jnp.float32)
    return (loss_total, loss_cls, loss_ver, loss_refine)
